# baseline (speedup 1.0000x reference)
import functools

import jax
import jax.numpy as jnp
from jax import lax
from jax.experimental import pallas as pl
from jax.experimental.pallas import tpu as pltpu
from jax.experimental.pallas import tpu_sc as plsc

EMBEDDING_DIM = 32
NUM_TABLE_ROWS = 1000000
NUM_CORES = 2
NUM_SUBCORES = 16
NUM_WORKERS = NUM_CORES * NUM_SUBCORES
CHUNK = 1280
N_BUF = 2

TBLK = 800
N_BLOCKS = NUM_TABLE_ROWS // TBLK


def _sc_transpose_table(table_t):
    mesh = plsc.VectorSubcoreMesh(core_axis_name="c", subcore_axis_name="s")
    base_n = N_BLOCKS // NUM_WORKERS
    rem = N_BLOCKS - base_n * NUM_WORKERS

    @functools.partial(
        pl.kernel,
        mesh=mesh,
        out_type=jax.ShapeDtypeStruct((NUM_TABLE_ROWS * EMBEDDING_DIM,), jnp.float32),
        scratch_types=[
            pltpu.VMEM((EMBEDDING_DIM * TBLK,), jnp.float32),
            pltpu.VMEM((TBLK * EMBEDDING_DIM,), jnp.float32),
            pltpu.SemaphoreType.DMA,
        ],
        compiler_params=pltpu.CompilerParams(
            use_tc_tiling_on_sc=False, needs_layout_passes=False
        ),
    )
    def k(tab_hbm, out_hbm, in_v, out_v, sem):
        wid = lax.axis_index("s") * NUM_CORES + lax.axis_index("c")
        n_mine = base_n + jnp.where(wid < rem, 1, 0)
        base_blk = wid * base_n + jnp.minimum(wid, rem)
        lane = lax.iota(jnp.int32, 16)
        base0 = lane * TBLK
        base16 = (lane + 16) * TBLK

        def do_block(blk, _):
            for d in range(EMBEDDING_DIM):
                pltpu.async_copy(
                    tab_hbm.at[d, pl.ds(blk * TBLK, TBLK)],
                    in_v.at[pl.ds(d * TBLK, TBLK)],
                    sem,
                )
            for d in range(EMBEDDING_DIM):
                pltpu.make_async_copy(
                    tab_hbm.at[0, pl.ds(0, TBLK)],
                    in_v.at[pl.ds(0, TBLK)],
                    sem,
                ).wait()

            def row(i, _):
                for d0, base in ((0, base0), (16, base16)):
                    vals = plsc.load_gather(in_v, [base + i])
                    out_v[pl.ds(i * EMBEDDING_DIM + d0, 16)] = vals
                return 0

            lax.fori_loop(0, TBLK, row, 0)
            pltpu.sync_copy(
                out_v,
                out_hbm.at[pl.ds(blk * TBLK * EMBEDDING_DIM, TBLK * EMBEDDING_DIM)],
            )
            return 0

        lax.fori_loop(base_blk, base_blk + n_mine, do_block, 0)

    return k(table_t)


def _sc_gather(table, idx_flat, n_total):
    b_per_w = n_total // NUM_WORKERS
    n_chunks = b_per_w // CHUNK
    n_outer = n_chunks // N_BUF
    mesh = plsc.VectorSubcoreMesh(core_axis_name="c", subcore_axis_name="s")

    @functools.partial(
        pl.kernel,
        mesh=mesh,
        out_type=jax.ShapeDtypeStruct((n_total, EMBEDDING_DIM), jnp.float32),
        scratch_types=[
            pltpu.VMEM((b_per_w,), jnp.int32),
            [pltpu.VMEM((CHUNK, EMBEDDING_DIM), jnp.float32) for _ in range(N_BUF)],
            [pltpu.SemaphoreType.DMA for _ in range(N_BUF)],
            [pltpu.SemaphoreType.DMA for _ in range(N_BUF)],
        ],
        compiler_params=pltpu.CompilerParams(use_tc_tiling_on_sc=False),
    )
    def k(table_hbm, idx_hbm, out_hbm, idx_v, rows, gsems, wsems):
        wid = lax.axis_index("s") * NUM_CORES + lax.axis_index("c")
        base = wid * b_per_w
        pltpu.sync_copy(idx_hbm.at[pl.ds(base, b_per_w)], idx_v)

        def g_start(c, p):
            pltpu.async_copy(
                table_hbm.at[idx_v.at[pl.ds(c * CHUNK, CHUNK)]], rows[p], gsems[p]
            )

        def g_wait(p):
            pltpu.make_async_copy(
                table_hbm.at[idx_v.at[pl.ds(0, CHUNK)]], rows[p], gsems[p]
            ).wait()

        def w_start(c, p):
            pltpu.async_copy(
                rows[p], out_hbm.at[pl.ds(base + c * CHUNK, CHUNK)], wsems[p]
            )

        def w_wait(p):
            pltpu.make_async_copy(
                rows[p], out_hbm.at[pl.ds(base, CHUNK)], wsems[p]
            ).wait()

        for p in range(N_BUF):
            g_start(p, p)

        def body(j, carry):
            for p in range(N_BUF):
                c = j * N_BUF + p
                g_wait(p)
                w_start(c, p)
                w_wait(p)
                g_start(c + N_BUF, p)
            return carry

        lax.fori_loop(0, n_outer - 1, body, 0)

        for p in range(N_BUF):
            g_wait(p)
            w_start((n_outer - 1) * N_BUF + p, p)
        for p in range(N_BUF):
            w_wait(p)

    return k(table, idx_flat)


def kernel(idx_texts, table):
    n_total = idx_texts.shape[0] * idx_texts.shape[1]
    idx_flat = idx_texts.reshape(-1).astype(jnp.int32)
    table_lin = _sc_transpose_table(table.T)
    table_l = table_lin.reshape(NUM_TABLE_ROWS, EMBEDDING_DIM)
    out = _sc_gather(table_l, idx_flat, n_total)
    return out.reshape(idx_texts.shape[0], idx_texts.shape[1], EMBEDDING_DIM)


# speedup vs baseline: 3.9471x; 3.9471x over previous
import functools

import jax
import jax.numpy as jnp
from jax import lax
from jax.experimental import pallas as pl
from jax.experimental.pallas import tpu as pltpu
from jax.experimental.pallas import tpu_sc as plsc

EMBEDDING_DIM = 32
NUM_CORES = 2
NUM_SUBCORES = 16
NUM_WORKERS = NUM_CORES * NUM_SUBCORES
CHUNK = 1280
N_BUF = 2


def _sc_gather(table, idx_flat, n_total):
    b_per_w = n_total // NUM_WORKERS
    n_chunks = b_per_w // CHUNK
    n_outer = n_chunks // N_BUF
    mesh = plsc.VectorSubcoreMesh(core_axis_name="c", subcore_axis_name="s")

    @functools.partial(
        pl.kernel,
        mesh=mesh,
        out_type=jax.ShapeDtypeStruct((n_total, EMBEDDING_DIM), jnp.float32),
        scratch_types=[
            pltpu.VMEM((b_per_w,), jnp.int32),
            [pltpu.VMEM((CHUNK, EMBEDDING_DIM), jnp.float32) for _ in range(N_BUF)],
            [pltpu.SemaphoreType.DMA for _ in range(N_BUF)],
            [pltpu.SemaphoreType.DMA for _ in range(N_BUF)],
        ],
        compiler_params=pltpu.CompilerParams(use_tc_tiling_on_sc=False),
    )
    def k(table_hbm, idx_hbm, out_hbm, idx_v, rows, gsems, wsems):
        wid = lax.axis_index("s") * NUM_CORES + lax.axis_index("c")
        base = wid * b_per_w
        pltpu.sync_copy(idx_hbm.at[pl.ds(base, b_per_w)], idx_v)

        def g_start(c, p):
            pltpu.async_copy(
                table_hbm.at[idx_v.at[pl.ds(c * CHUNK, CHUNK)]], rows[p], gsems[p]
            )

        def g_wait(p):
            pltpu.make_async_copy(
                table_hbm.at[idx_v.at[pl.ds(0, CHUNK)]], rows[p], gsems[p]
            ).wait()

        def w_start(c, p):
            pltpu.async_copy(
                rows[p], out_hbm.at[pl.ds(base + c * CHUNK, CHUNK)], wsems[p]
            )

        def w_wait(p):
            pltpu.make_async_copy(
                rows[p], out_hbm.at[pl.ds(base, CHUNK)], wsems[p]
            ).wait()

        for p in range(N_BUF):
            g_start(p, p)

        def body(j, carry):
            for p in range(N_BUF):
                c = j * N_BUF + p
                g_wait(p)
                w_start(c, p)
                w_wait(p)
                g_start(c + N_BUF, p)
            return carry

        lax.fori_loop(0, n_outer - 1, body, 0)

        for p in range(N_BUF):
            g_wait(p)
            w_start((n_outer - 1) * N_BUF + p, p)
        for p in range(N_BUF):
            w_wait(p)

    return k(table, idx_flat)


def kernel(idx_texts, table):
    n_total = idx_texts.shape[0] * idx_texts.shape[1]
    idx_flat = idx_texts.reshape(-1).astype(jnp.int32)
    out = _sc_gather(table, idx_flat, n_total)
    return out.reshape(idx_texts.shape[0], idx_texts.shape[1], EMBEDDING_DIM)
